# baseline (speedup 1.0000x reference)
import functools

import jax
import jax.numpy as jnp
from jax import lax
from jax.experimental import pallas as pl
from jax.experimental.pallas import tpu as pltpu
from jax.experimental.pallas import tpu_sc as plsc

D_MODEL = 128
CHUNK = 128
NBUF = 5


@functools.lru_cache(maxsize=None)
def _make_kernel(b_flat: int, vocab: int):
    info = plsc.get_sparse_core_info()
    nc, ns = info.num_cores, info.num_subcores
    nw = nc * ns
    b_per_w = b_flat // nw
    n_chunks = b_per_w // CHUNK
    n_groups = (n_chunks + NBUF - 1) // NBUF
    mesh = plsc.VectorSubcoreMesh(core_axis_name="c", subcore_axis_name="s")

    @functools.partial(
        pl.kernel,
        mesh=mesh,
        out_type=jax.ShapeDtypeStruct((b_flat, D_MODEL), jnp.float32),
        scratch_types=(
            [pltpu.VMEM((b_per_w,), jnp.int32)]
            + [pltpu.VMEM((CHUNK, D_MODEL), jnp.float32)] * NBUF
            + [pltpu.SemaphoreType.DMA] * (2 * NBUF)
        ),
    )
    def gather_kernel(idx_hbm, table_hbm, out_hbm, idx_v, *rest):
        bufs = rest[:NBUF]
        gsems = rest[NBUF : 2 * NBUF]
        wsems = rest[2 * NBUF :]

        wid = lax.axis_index("s") * nc + lax.axis_index("c")
        base = wid * b_per_w
        pltpu.sync_copy(idx_hbm.at[pl.ds(base, b_per_w)], idx_v)

        def start_gather(j, b_):
            pltpu.async_copy(
                table_hbm.at[idx_v.at[pl.ds(j * CHUNK, CHUNK)]], bufs[b_], gsems[b_]
            )

        def wait_gather(b_):
            pltpu.make_async_copy(
                out_hbm.at[pl.ds(0, CHUNK)], bufs[b_], gsems[b_]
            ).wait()

        def start_wb(j, b_):
            pltpu.async_copy(
                bufs[b_], out_hbm.at[pl.ds(base + j * CHUNK, CHUNK)], wsems[b_]
            )

        def wait_wb(b_):
            pltpu.make_async_copy(
                bufs[b_], out_hbm.at[pl.ds(0, CHUNK)], wsems[b_]
            ).wait()

        for b_ in range(NBUF):
            start_gather(b_, b_)

        def group(g, carry):
            for b_ in range(NBUF):
                j = NBUF * g + b_
                wait_gather(b_)
                start_wb(j, b_)

                pb = (b_ - 1) % NBUF

                def recycle(jprev):
                    @pl.when(jprev + NBUF < n_chunks)
                    def _():
                        wait_wb(pb)
                        start_gather(jprev + NBUF, pb)

                if b_ == 0:
                    @pl.when(g > 0)
                    def _():
                        recycle(j - 1)
                else:
                    recycle(j - 1)

            return carry

        lax.fori_loop(0, n_groups, group, 0)

        for b_ in range(NBUF):
            wait_wb(b_)

    return gather_kernel


def kernel(token_ids, table):
    b, s = token_ids.shape
    vocab, d = table.shape
    b_flat = b * s
    idx = jnp.asarray(token_ids, jnp.int32).T.reshape(b_flat)
    out = _make_kernel(b_flat, vocab)(idx, table)
    return out.reshape(s, b, d).transpose(1, 0, 2)
